# baseline (speedup 1.0000x reference)
import jax
import jax.numpy as jnp
from jax.experimental import pallas as pl


def _add_body(enc_ref, pos_ref, out_ref):
    out_ref[...] = enc_ref[...] + pos_ref[...]


def kernel(encoded_patches, pos_table):
    B, P, D = encoded_patches.shape
    return pl.pallas_call(
        _add_body,
        grid=(B,),
        in_specs=[
            pl.BlockSpec((1, P, D), lambda b: (b, 0, 0)),
            pl.BlockSpec((P, D), lambda b: (0, 0)),
        ],
        out_specs=pl.BlockSpec((1, P, D), lambda b: (b, 0, 0)),
        out_shape=jax.ShapeDtypeStruct((B, P, D), jnp.float32),
    )(encoded_patches, pos_table)
